# baseline (speedup 1.0000x reference)
import functools

import jax
import jax.numpy as jnp
from jax import lax
from jax.experimental import pallas as pl
from jax.experimental.pallas import tpu as pltpu
from jax.experimental.pallas import tpu_sc as plsc

D_MODEL = 64
SCALE = 8.0
NUM_WORKERS = 32
B_TOTAL = 4096 * 200
B_PER_W = B_TOTAL // NUM_WORKERS
STREAM = 128
K = 4
BUF = K * STREAM
N_CHUNKS = B_PER_W // BUF
IDX_ROWS_PER_W = B_PER_W // STREAM


def _emb_body(x_hbm, table_hbm, out_hbm, idx_v, rows_v, gsem, ssem):
    c = lax.axis_index("c")
    s = lax.axis_index("s")
    wid = s * 2 + c
    idx_base = wid * IDX_ROWS_PER_W
    out_base = wid * B_PER_W

    def stage_and_fire(g, b):
        pltpu.sync_copy(x_hbm.at[pl.ds(idx_base + g * K, K)], idx_v.at[b])
        for j in range(K):
            pltpu.async_copy(
                table_hbm.at[idx_v.at[b, j]],
                rows_v.at[b, pl.ds(j * STREAM, STREAM)],
                gsem.at[b],
            )

    def drain_gathers(b):
        for j in range(K):
            pltpu.make_async_copy(
                table_hbm.at[idx_v.at[b, j]],
                rows_v.at[b, pl.ds(j * STREAM, STREAM)],
                gsem.at[b],
            ).wait()

    stage_and_fire(0, 0)

    @pl.loop(0, N_CHUNKS, step=2)
    def _chunk_pair(t):
        for b in range(2):
            g = t + b
            nb = 1 - b

            @pl.when(g >= 1)
            def _():
                pltpu.make_async_copy(
                    rows_v.at[nb], out_hbm.at[pl.ds(0, BUF)], ssem.at[nb]
                ).wait()

            @pl.when(g + 1 < N_CHUNKS)
            def _():
                stage_and_fire(g + 1, nb)

            drain_gathers(b)

            @plsc.parallel_loop(0, BUF, unroll=8)
            def _scale(i):
                for k4 in range(D_MODEL // 16):
                    sl = rows_v[b, i, pl.ds(k4 * 16, 16)]
                    rows_v[b, i, pl.ds(k4 * 16, 16)] = sl * SCALE

            pltpu.async_copy(
                rows_v.at[b],
                out_hbm.at[pl.ds(out_base + g * BUF, BUF)],
                ssem.at[b],
            )

    pltpu.make_async_copy(
        rows_v.at[1], out_hbm.at[pl.ds(0, BUF)], ssem.at[1]
    ).wait()


@jax.jit
def _emb(x2d, table):
    mesh = plsc.VectorSubcoreMesh(core_axis_name="c", subcore_axis_name="s")
    f = pl.kernel(
        _emb_body,
        mesh=mesh,
        out_type=jax.ShapeDtypeStruct((B_TOTAL, D_MODEL), jnp.float32),
        scratch_types=[
            pltpu.VMEM((2, K, STREAM), jnp.int32),
            pltpu.VMEM((2, BUF, D_MODEL), jnp.float32),
            pltpu.SemaphoreType.DMA((2,)),
            pltpu.SemaphoreType.DMA((2,)),
        ],
        compiler_params=pltpu.CompilerParams(use_tc_tiling_on_sc=False),
    )
    return f(x2d, table)


def kernel(x, table):
    x2d = x.reshape(B_TOTAL // STREAM, STREAM).astype(jnp.int32)
    out = _emb(x2d, table)
    return out.reshape(x.shape[0], x.shape[1], D_MODEL)
